# baseline (speedup 1.0000x reference)
import functools

import jax
import jax.numpy as jnp
from jax import lax
from jax.experimental import pallas as pl
from jax.experimental.pallas import tpu as pltpu
from jax.experimental.pallas import tpu_sc as plsc

_BATCH = 16384
_FEAT = 64
_NC = 2
_NS = 16
_NW = _NC * _NS
_BPW = _BATCH // _NW
_CHUNK = 128
_NCHUNK = _BPW // _CHUNK
_LANES = 16

_mesh = plsc.VectorSubcoreMesh(core_axis_name="c", subcore_axis_name="s")


@functools.partial(
    pl.kernel,
    mesh=_mesh,
    compiler_params=pltpu.CompilerParams(use_tc_tiling_on_sc=False),
    out_type=jax.ShapeDtypeStruct((_NW, _LANES), jnp.float32),
    scratch_types=[
        pltpu.VMEM((_NCHUNK, _CHUNK), jnp.int32),
        pltpu.VMEM((_BPW, _FEAT), jnp.float32),
        pltpu.VMEM((_BPW, _FEAT), jnp.float32),
        pltpu.VMEM((_LANES,), jnp.float32),
        pltpu.SemaphoreType.DMA,
        pltpu.SemaphoreType.DMA,
    ],
)
def _center_loss_sc(feat_hbm, lab_hbm, cent_hbm, out_hbm,
                    idx_v, cen_v, feat_v, acc_v, gsem, fsem):
    wid = lax.axis_index("s") * _NC + lax.axis_index("c")
    base = wid * _BPW

    for j in range(_NCHUNK):
        pltpu.sync_copy(lab_hbm.at[pl.ds(base + j * _CHUNK, _CHUNK)],
                        idx_v.at[j])

    fcopy = pltpu.async_copy(feat_hbm.at[pl.ds(base, _BPW)], feat_v, fsem)
    gcopies = [
        pltpu.async_copy(cent_hbm.at[idx_v.at[j]],
                         cen_v.at[pl.ds(j * _CHUNK, _CHUNK)], gsem)
        for j in range(_NCHUNK)
    ]
    fcopy.wait()
    for c in gcopies:
        c.wait()

    def body(i, acc):
        for j in range(_FEAT // _LANES):
            d = (feat_v[i, pl.ds(j * _LANES, _LANES)]
                 - cen_v[i, pl.ds(j * _LANES, _LANES)])
            acc = acc + d * d
        return acc

    acc = lax.fori_loop(0, _BPW, body, jnp.zeros((_LANES,), jnp.float32))
    acc_v[...] = acc * (1.0 / (2.0 * _BATCH))
    pltpu.sync_copy(acc_v, out_hbm.at[wid])


def kernel(features, labels, centers):
    partials = _center_loss_sc(features, labels.astype(jnp.int32), centers)
    return jnp.sum(partials)
